# Initial kernel scaffold; baseline (speedup 1.0000x reference)
import jax
import jax.numpy as jnp
from jax.experimental import pallas as pl


def kernel(x, mask, gamma, beta):
    raise NotImplementedError("write your pallas kernel here")


# speedup vs baseline: 1.1652x; 1.1652x over previous
import jax
import jax.numpy as jnp
from jax.experimental import pallas as pl

EPS_ = 1e-5


def _stats_body(x_ref, m_ref, o_ref):
    @pl.when(pl.program_id(0) == 0)
    def _init():
        o_ref[...] = jnp.zeros_like(o_ref)

    xv = x_ref[...]
    mv = m_ref[...]
    xm = xv * mv
    s = jnp.sum(xm, axis=0, keepdims=True)
    s2 = jnp.sum(xm * xv, axis=0, keepdims=True)
    cnt = jnp.sum(mv)
    d = o_ref.shape[1]
    upd = jnp.concatenate([s, s2, jnp.full((1, d), cnt, jnp.float32)], axis=0)
    o_ref[...] += upd


def _norm_body(x_ref, m_ref, st_ref, g_ref, b_ref, o_ref):
    n = st_ref[2:3, 0:1]
    mean = st_ref[0:1, :] / n
    var = st_ref[1:2, :] / n - mean * mean
    inv = jax.lax.rsqrt(var + EPS_)
    scale = g_ref[...] * inv
    bias = b_ref[...] - mean * scale
    xv = x_ref[...]
    o_ref[...] = jnp.where(m_ref[...] > 0, xv * scale + bias, xv)


def kernel(x, mask, gamma, beta):
    b, s, d = x.shape
    n = b * s
    xf = x.reshape(n, d)
    m = mask.reshape(n, 1).astype(jnp.float32)
    bt = 2048
    grid = (n // bt,)

    stats = pl.pallas_call(
        _stats_body,
        grid=grid,
        in_specs=[
            pl.BlockSpec((bt, d), lambda i: (i, 0)),
            pl.BlockSpec((bt, 1), lambda i: (i, 0)),
        ],
        out_specs=pl.BlockSpec((3, d), lambda i: (0, 0)),
        out_shape=jax.ShapeDtypeStruct((3, d), jnp.float32),
    )(xf, m)

    out = pl.pallas_call(
        _norm_body,
        grid=grid,
        in_specs=[
            pl.BlockSpec((bt, d), lambda i: (i, 0)),
            pl.BlockSpec((bt, 1), lambda i: (i, 0)),
            pl.BlockSpec((3, d), lambda i: (0, 0)),
            pl.BlockSpec((1, d), lambda i: (0, 0)),
            pl.BlockSpec((1, d), lambda i: (0, 0)),
        ],
        out_specs=pl.BlockSpec((bt, d), lambda i: (i, 0)),
        out_shape=jax.ShapeDtypeStruct((n, d), jnp.float32),
    )(xf, m, stats, gamma.reshape(1, d), beta.reshape(1, d))

    return out.reshape(b, s, d)


# speedup vs baseline: 1.1926x; 1.0235x over previous
import jax
import jax.numpy as jnp
from jax.experimental import pallas as pl

EPS_ = 1e-5


def _stats_body(x_ref, mr_ref, o_ref):
    @pl.when(pl.program_id(0) == 0)
    def _init():
        o_ref[...] = jnp.zeros_like(o_ref)

    xv = x_ref[...]
    mr = mr_ref[...]
    s = jax.lax.dot_general(mr, xv, (((1,), (0,)), ((), ())),
                            preferred_element_type=jnp.float32)
    s2 = jax.lax.dot_general(mr, xv * xv, (((1,), (0,)), ((), ())),
                             preferred_element_type=jnp.float32)
    cnt = jnp.sum(mr)
    d = o_ref.shape[1]
    upd = jnp.concatenate([s, s2, jnp.full((1, d), cnt, jnp.float32)], axis=0)
    o_ref[...] += upd


def _norm_body(x_ref, m_ref, st_ref, g_ref, b_ref, o_ref):
    n = st_ref[2:3, 0:1]
    mean = st_ref[0:1, :] / n
    var = st_ref[1:2, :] / n - mean * mean
    inv = jax.lax.rsqrt(var + EPS_)
    scale = g_ref[...] * inv
    bias = b_ref[...] - mean * scale
    xv = x_ref[...]
    o_ref[...] = jnp.where(m_ref[...] > 0, xv * scale + bias, xv)


def kernel(x, mask, gamma, beta):
    b, s, d = x.shape
    n = b * s
    xf = x.reshape(n, d)
    m = mask.reshape(n, 1).astype(jnp.float32)
    m_row = mask.reshape(1, n).astype(jnp.float32)
    bt = 2048
    grid = (n // bt,)

    stats = pl.pallas_call(
        _stats_body,
        grid=grid,
        in_specs=[
            pl.BlockSpec((bt, d), lambda i: (i, 0)),
            pl.BlockSpec((1, bt), lambda i: (0, i)),
        ],
        out_specs=pl.BlockSpec((3, d), lambda i: (0, 0)),
        out_shape=jax.ShapeDtypeStruct((3, d), jnp.float32),
    )(xf, m_row)

    out = pl.pallas_call(
        _norm_body,
        grid=grid,
        in_specs=[
            pl.BlockSpec((bt, d), lambda i: (i, 0)),
            pl.BlockSpec((bt, 1), lambda i: (i, 0)),
            pl.BlockSpec((3, d), lambda i: (0, 0)),
            pl.BlockSpec((1, d), lambda i: (0, 0)),
            pl.BlockSpec((1, d), lambda i: (0, 0)),
        ],
        out_specs=pl.BlockSpec((bt, d), lambda i: (i, 0)),
        out_shape=jax.ShapeDtypeStruct((n, d), jnp.float32),
    )(xf, m, stats, gamma.reshape(1, d), beta.reshape(1, d))

    return out.reshape(b, s, d)
